# baseline (speedup 1.0000x reference)
import jax
import jax.numpy as jnp
from jax import lax
from jax.experimental import pallas as pl
from jax.experimental.pallas import tpu as pltpu
from jax.experimental.pallas import tpu_sc as plsc

_B, _L, _D = 4096, 200, 64
_NC, _NS = 2, 16
_LANES = 16
_CW = 80

_SC_PARAMS = pltpu.CompilerParams(
    use_tc_tiling_on_sc=False, needs_layout_passes=False)


def _mesh(nc, ns):
    return plsc.VectorSubcoreMesh(
        core_axis_name="c", subcore_axis_name="s",
        num_cores=nc, num_subcores=ns)


def _build_history(B, L, D, nc, ns, chunk_w):
    nw = nc * ns
    bpw = B // nw
    R = 2 * L
    n_ch = R // chunk_w
    nvec = (L + _LANES - 1) // _LANES
    lpad = nvec * _LANES
    assert R % chunk_w == 0 and chunk_w % 8 == 0 and chunk_w <= 128
    assert B % nw == 0 and bpw % 8 == 0 and D % _LANES == 0 and bpw % 2 == 0
    nv = D // _LANES

    def body(ih_i, ch_i, item_t,
             his_o, sum_o,
             sum_v,
             ia_a, ic_a, hidx_a, rows_a,
             ia_b, ic_b, hidx_b, rows_b,
             gsem_a, gsem_b, ssem_a, ssem_b, isem_a, isem_b):
        wid = lax.axis_index("s") * nc + lax.axis_index("c")
        base = pl.multiple_of(wid * bpw, 8)

        iota = lax.iota(jnp.int32, _LANES)
        tail_mask = iota < (L - (nvec - 1) * _LANES)

        def fetch(b, ia, ic, isem):
            pltpu.async_copy(ih_i.at[b], ia.at[pl.ds(0, L)], isem)
            pltpu.async_copy(ch_i.at[b], ic.at[pl.ds(0, L)], isem)

        def drain_fetch(ia, ic, isem):
            pltpu.make_async_copy(ih_i.at[0], ia.at[pl.ds(0, L)], isem).wait()
            pltpu.make_async_copy(ch_i.at[0], ic.at[pl.ds(0, L)], isem).wait()

        def interleave(ia, ic, hidx):
            for j in range(nvec):
                pos = (iota + (j * _LANES)) * 2
                va = ia[pl.ds(j * _LANES, _LANES)]
                vc = ic[pl.ds(j * _LANES, _LANES)]
                m = None if j < nvec - 1 else tail_mask
                plsc.store_scatter(hidx, [pos], va, mask=m)
                plsc.store_scatter(hidx, [pos + 1], vc, mask=m)

        def fire(b, ia, ic, hidx, rows, isem, gsem):
            drain_fetch(ia, ic, isem)
            interleave(ia, ic, hidx)
            for k in range(n_ch):
                pltpu.async_copy(item_t.at[hidx.at[pl.ds(k * chunk_w, chunk_w)]],
                                 rows.at[pl.ds(k * chunk_w, chunk_w)], gsem)

        def drain_gather(hidx, rows, gsem):
            for k in range(n_ch):
                pltpu.make_async_copy(
                    item_t.at[hidx.at[pl.ds(0, chunk_w)]],
                    rows.at[pl.ds(k * chunk_w, chunk_w)], gsem).wait()

        def drain_store(rows, ssem):
            pltpu.make_async_copy(rows, his_o.at[pl.ds(0, R)], ssem).wait()

        def consume(i, b, hidx, rows, gsem, ssem):
            drain_gather(hidx, rows, gsem)

            def lstep(l, acc):
                out = []
                for k in range(nv):
                    out.append(acc[k] + rows[2 * l, pl.ds(k * _LANES, _LANES)])
                for k in range(nv):
                    out.append(acc[nv + k]
                               + rows[2 * l + 1, pl.ds(k * _LANES, _LANES)])
                return tuple(out)

            zero = jnp.zeros((_LANES,), jnp.float32)
            acc = lax.fori_loop(0, L, lstep, (zero,) * (2 * nv))
            for k in range(2 * nv):
                sum_v[i, pl.ds(k * _LANES, _LANES)] = acc[k]
            off = pl.multiple_of(b * R, 8)
            pltpu.async_copy(rows, his_o.at[pl.ds(off, R)], ssem)

        fetch(base + 0, ia_a, ic_a, isem_a)
        fetch(base + 1, ia_b, ic_b, isem_b)
        fire(base + 0, ia_a, ic_a, hidx_a, rows_a, isem_a, gsem_a)

        def tstep(t, carry):
            b0 = base + 2 * t
            fire(b0 + 1, ia_b, ic_b, hidx_b, rows_b, isem_b, gsem_b)

            @pl.when(t < bpw // 2 - 1)
            def _():
                fetch(b0 + 2, ia_a, ic_a, isem_a)

            consume(2 * t, b0, hidx_a, rows_a, gsem_a, ssem_a)

            @pl.when(t < bpw // 2 - 1)
            def _():
                fetch(b0 + 3, ia_b, ic_b, isem_b)
                drain_store(rows_a, ssem_a)
                fire(b0 + 2, ia_a, ic_a, hidx_a, rows_a, isem_a, gsem_a)

            consume(2 * t + 1, b0 + 1, hidx_b, rows_b, gsem_b, ssem_b)

            @pl.when(t < bpw // 2 - 1)
            def _():
                drain_store(rows_b, ssem_b)

            return carry

        lax.fori_loop(0, bpw // 2, tstep, 0)
        drain_store(rows_a, ssem_a)
        drain_store(rows_b, ssem_b)
        pltpu.sync_copy(sum_v, sum_o.at[pl.ds(base, bpw)])

    return pl.kernel(
        body,
        out_type=(
            jax.ShapeDtypeStruct((B * R, D), jnp.float32),
            jax.ShapeDtypeStruct((B, 2 * D), jnp.float32),
        ),
        mesh=_mesh(nc, ns),
        scratch_types=(
            pltpu.VMEM((bpw, 2 * D), jnp.float32),
            pltpu.VMEM((lpad,), jnp.int32),
            pltpu.VMEM((lpad,), jnp.int32),
            pltpu.VMEM((R,), jnp.int32),
            pltpu.VMEM((R, D), jnp.float32),
            pltpu.VMEM((lpad,), jnp.int32),
            pltpu.VMEM((lpad,), jnp.int32),
            pltpu.VMEM((R,), jnp.int32),
            pltpu.VMEM((R, D), jnp.float32),
            pltpu.SemaphoreType.DMA,
            pltpu.SemaphoreType.DMA,
            pltpu.SemaphoreType.DMA,
            pltpu.SemaphoreType.DMA,
            pltpu.SemaphoreType.DMA,
            pltpu.SemaphoreType.DMA,
        ),
        compiler_params=_SC_PARAMS,
    )


def _build_singles(B, D, nc, ns):
    nw = nc * ns
    bpw = B // nw
    assert B % nw == 0 and bpw % 8 == 0

    def body(user_i, item_i, cate_i, user_t, item_t, cate_t,
             user_o, joina_o, joinb_o,
             sidx_v, small_v, sem0):
        wid = lax.axis_index("s") * nc + lax.axis_index("c")
        base = pl.multiple_of(wid * bpw, 8)

        pltpu.sync_copy(user_i.at[pl.ds(base, bpw)], sidx_v)
        pltpu.async_copy(user_t.at[sidx_v], small_v, sem0).wait()
        pltpu.sync_copy(small_v, user_o.at[pl.ds(base, bpw)])

        pltpu.sync_copy(item_i.at[pl.ds(base, bpw)], sidx_v)
        pltpu.async_copy(item_t.at[sidx_v], small_v, sem0).wait()
        pltpu.sync_copy(small_v, joina_o.at[pl.ds(base, bpw)])

        pltpu.sync_copy(cate_i.at[pl.ds(base, bpw)], sidx_v)
        pltpu.async_copy(cate_t.at[sidx_v], small_v, sem0).wait()
        pltpu.sync_copy(small_v, joinb_o.at[pl.ds(base, bpw)])

    return pl.kernel(
        body,
        out_type=(
            jax.ShapeDtypeStruct((B, D), jnp.float32),
            jax.ShapeDtypeStruct((B, D), jnp.float32),
            jax.ShapeDtypeStruct((B, D), jnp.float32),
        ),
        mesh=_mesh(nc, ns),
        scratch_types=(
            pltpu.VMEM((bpw,), jnp.int32),
            pltpu.VMEM((bpw, D), jnp.float32),
            pltpu.SemaphoreType.DMA,
        ),
        compiler_params=_SC_PARAMS,
    )


@jax.jit
def _run(user_i, item_i, cate_i, ih_i, ch_i, user_t, item_t, cate_t):
    his_flat, his_sum = _build_history(_B, _L, _D, _NC, _NS, chunk_w=_CW)(
        ih_i, ch_i, item_t)
    user_emb, join_a, join_b = _build_singles(_B, _D, _NC, _NS)(
        user_i, item_i, cate_i, user_t, item_t, cate_t)
    return user_emb, join_a, join_b, his_flat, his_sum


def kernel(user, item, cate, item_his, cate_his, user_table, item_table,
           cate_table):
    i32 = jnp.int32
    user_emb, join_a, join_b, his_flat, his_sum = _run(
        user.astype(i32), item.astype(i32), cate.astype(i32),
        item_his.astype(i32), cate_his.astype(i32),
        user_table, item_table, cate_table)
    join_emb = jnp.concatenate([join_a, join_b], axis=-1)
    return (user_emb, join_emb,
            his_flat.reshape(_B, _L, 2 * _D), his_sum)


# speedup vs baseline: 1.0765x; 1.0765x over previous
import jax
import jax.numpy as jnp
from jax import lax
from jax.experimental import pallas as pl
from jax.experimental.pallas import tpu as pltpu
from jax.experimental.pallas import tpu_sc as plsc

_B, _L, _D = 4096, 200, 64
_NC, _NS = 2, 16
_LANES = 16
_CW = 80

_SC_PARAMS = pltpu.CompilerParams(
    use_tc_tiling_on_sc=False, needs_layout_passes=False)


def _mesh(nc, ns):
    return plsc.VectorSubcoreMesh(
        core_axis_name="c", subcore_axis_name="s",
        num_cores=nc, num_subcores=ns)


def _build_history(B, L, D, nc, ns, chunk_w):
    nw = nc * ns
    bpw = B // nw
    R = 2 * L
    n_ch = R // chunk_w
    nvec = (L + _LANES - 1) // _LANES
    lpad = nvec * _LANES
    assert R % chunk_w == 0 and chunk_w % 8 == 0 and chunk_w <= 128
    assert B % nw == 0 and bpw % 8 == 0 and D % _LANES == 0 and bpw % 2 == 0
    nv = D // _LANES

    def body(ih_i, ch_i, item_t,
             his_o, sum_o,
             sum_v,
             ia_a, ic_a, hidx_a, rows_a,
             ia_b, ic_b, hidx_b, rows_b,
             gsem_a, gsem_b, ssem_a, ssem_b, isem_a, isem_b):
        wid = lax.axis_index("s") * nc + lax.axis_index("c")
        base = pl.multiple_of(wid * bpw, 8)

        iota = lax.iota(jnp.int32, _LANES)
        tail_mask = iota < (L - (nvec - 1) * _LANES)

        def fetch(b, ia, ic, isem):
            pltpu.async_copy(ih_i.at[b], ia.at[pl.ds(0, L)], isem)
            pltpu.async_copy(ch_i.at[b], ic.at[pl.ds(0, L)], isem)

        def drain_fetch(ia, ic, isem):
            pltpu.make_async_copy(ih_i.at[0], ia.at[pl.ds(0, L)], isem).wait()
            pltpu.make_async_copy(ch_i.at[0], ic.at[pl.ds(0, L)], isem).wait()

        def interleave(ia, ic, hidx):
            for j in range(nvec):
                pos = (iota + (j * _LANES)) * 2
                va = ia[pl.ds(j * _LANES, _LANES)]
                vc = ic[pl.ds(j * _LANES, _LANES)]
                m = None if j < nvec - 1 else tail_mask
                plsc.store_scatter(hidx, [pos], va, mask=m)
                plsc.store_scatter(hidx, [pos + 1], vc, mask=m)

        def fire(b, ia, ic, hidx, rows, isem, gsem):
            drain_fetch(ia, ic, isem)
            interleave(ia, ic, hidx)
            for k in range(n_ch):
                pltpu.async_copy(item_t.at[hidx.at[pl.ds(k * chunk_w, chunk_w)]],
                                 rows.at[pl.ds(k * chunk_w, chunk_w)], gsem)

        def drain_gather(hidx, rows, gsem):
            for k in range(n_ch):
                pltpu.make_async_copy(
                    item_t.at[hidx.at[pl.ds(0, chunk_w)]],
                    rows.at[pl.ds(k * chunk_w, chunk_w)], gsem).wait()

        def drain_store(rows, ssem):
            pltpu.make_async_copy(rows, his_o.at[pl.ds(0, R)], ssem).wait()

        def consume(i, b, hidx, rows, gsem, ssem):
            drain_gather(hidx, rows, gsem)

            def lstep(l, acc):
                out = []
                for k in range(nv):
                    out.append(acc[k] + rows[2 * l, pl.ds(k * _LANES, _LANES)])
                for k in range(nv):
                    out.append(acc[nv + k]
                               + rows[2 * l + 1, pl.ds(k * _LANES, _LANES)])
                return tuple(out)

            zero = jnp.zeros((_LANES,), jnp.float32)
            acc = lax.fori_loop(0, L, lstep, (zero,) * (2 * nv))
            for k in range(2 * nv):
                sum_v[i, pl.ds(k * _LANES, _LANES)] = acc[k]
            off = pl.multiple_of(b * R, 8)
            pltpu.async_copy(rows, his_o.at[pl.ds(off, R)], ssem)

        fetch(base + 0, ia_a, ic_a, isem_a)
        fetch(base + 1, ia_b, ic_b, isem_b)
        fire(base + 0, ia_a, ic_a, hidx_a, rows_a, isem_a, gsem_a)

        def tstep(t, carry):
            b0 = base + 2 * t
            fire(b0 + 1, ia_b, ic_b, hidx_b, rows_b, isem_b, gsem_b)

            @pl.when(t < bpw // 2 - 1)
            def _():
                fetch(b0 + 2, ia_a, ic_a, isem_a)

            consume(2 * t, b0, hidx_a, rows_a, gsem_a, ssem_a)

            @pl.when(t < bpw // 2 - 1)
            def _():
                fetch(b0 + 3, ia_b, ic_b, isem_b)
                drain_store(rows_a, ssem_a)
                fire(b0 + 2, ia_a, ic_a, hidx_a, rows_a, isem_a, gsem_a)

            consume(2 * t + 1, b0 + 1, hidx_b, rows_b, gsem_b, ssem_b)

            @pl.when(t < bpw // 2 - 1)
            def _():
                drain_store(rows_b, ssem_b)

            return carry

        lax.fori_loop(0, bpw // 2, tstep, 0)
        drain_store(rows_a, ssem_a)
        drain_store(rows_b, ssem_b)
        pltpu.sync_copy(sum_v, sum_o.at[pl.ds(base, bpw)])

    return pl.kernel(
        body,
        out_type=(
            jax.ShapeDtypeStruct((B * R, D), jnp.float32),
            jax.ShapeDtypeStruct((B, 2 * D), jnp.float32),
        ),
        mesh=_mesh(nc, ns),
        scratch_types=(
            pltpu.VMEM((bpw, 2 * D), jnp.float32),
            pltpu.VMEM((lpad,), jnp.int32),
            pltpu.VMEM((lpad,), jnp.int32),
            pltpu.VMEM((R,), jnp.int32),
            pltpu.VMEM((R, D), jnp.float32),
            pltpu.VMEM((lpad,), jnp.int32),
            pltpu.VMEM((lpad,), jnp.int32),
            pltpu.VMEM((R,), jnp.int32),
            pltpu.VMEM((R, D), jnp.float32),
            pltpu.SemaphoreType.DMA,
            pltpu.SemaphoreType.DMA,
            pltpu.SemaphoreType.DMA,
            pltpu.SemaphoreType.DMA,
            pltpu.SemaphoreType.DMA,
            pltpu.SemaphoreType.DMA,
        ),
        compiler_params=_SC_PARAMS,
    )


def _build_singles(B, D, nc, ns):
    nw = nc * ns
    bpw = B // nw
    assert B % nw == 0 and bpw % 8 == 0

    def body(user_i, item_i, cate_i, user_t, item_t, cate_t, dep,
             user_o, joina_o, joinb_o,
             sidx_v, small_v, sem0):
        del dep
        wid = lax.axis_index("s") * nc + lax.axis_index("c")
        base = pl.multiple_of(wid * bpw, 8)

        pltpu.sync_copy(user_i.at[pl.ds(base, bpw)], sidx_v)
        pltpu.async_copy(user_t.at[sidx_v], small_v, sem0).wait()
        pltpu.sync_copy(small_v, user_o.at[pl.ds(base, bpw)])

        pltpu.sync_copy(item_i.at[pl.ds(base, bpw)], sidx_v)
        pltpu.async_copy(item_t.at[sidx_v], small_v, sem0).wait()
        pltpu.sync_copy(small_v, joina_o.at[pl.ds(base, bpw)])

        pltpu.sync_copy(cate_i.at[pl.ds(base, bpw)], sidx_v)
        pltpu.async_copy(cate_t.at[sidx_v], small_v, sem0).wait()
        pltpu.sync_copy(small_v, joinb_o.at[pl.ds(base, bpw)])

    return pl.kernel(
        body,
        out_type=(
            jax.ShapeDtypeStruct((B, D), jnp.float32),
            jax.ShapeDtypeStruct((B, D), jnp.float32),
            jax.ShapeDtypeStruct((B, D), jnp.float32),
        ),
        mesh=_mesh(nc, ns),
        scratch_types=(
            pltpu.VMEM((bpw,), jnp.int32),
            pltpu.VMEM((bpw, D), jnp.float32),
            pltpu.SemaphoreType.DMA,
        ),
        compiler_params=_SC_PARAMS,
    )


@jax.jit
def _run(user_i, item_i, cate_i, ih_i, ch_i, user_t, item_t, cate_t):
    his_flat, his_sum = _build_history(_B, _L, _D, _NC, _NS, chunk_w=_CW)(
        ih_i, ch_i, item_t)
    user_emb, join_a, join_b = _build_singles(_B, _D, _NC, _NS)(
        user_i, item_i, cate_i, user_t, item_t, cate_t, his_sum)
    return user_emb, join_a, join_b, his_flat, his_sum


def kernel(user, item, cate, item_his, cate_his, user_table, item_table,
           cate_table):
    i32 = jnp.int32
    user_emb, join_a, join_b, his_flat, his_sum = _run(
        user.astype(i32), item.astype(i32), cate.astype(i32),
        item_his.astype(i32), cate_his.astype(i32),
        user_table, item_table, cate_table)
    join_emb = jnp.concatenate([join_a, join_b], axis=-1)
    return (user_emb, join_emb,
            his_flat.reshape(_B, _L, 2 * _D), his_sum)
